# baseline (speedup 1.0000x reference)
import functools

import jax
import jax.numpy as jnp
from jax import lax
from jax.experimental import pallas as pl
from jax.experimental.pallas import tpu as pltpu
from jax.experimental.pallas import tpu_sc as plsc

B = 4096
LA = 200
LS = 50
L = LA + LS
HALF = 125
D = 64
DH = 32
NC = 2
NS = 16
NW = NC * NS
BPW = B // NW

VB = 8192
NVB = 123
GB = VB // 8
TAB_ROWS = NVB * VB


def _bf16_bits(xf):
    r = jax.lax.bitcast_convert_type(xf, jnp.int32)
    return ((r + 0x7FFF + ((r >> 16) & 1)) >> 16) & 0xFFFF


def _lin_half_body(et_ref, o_ref):
    x = et_ref[...]
    lo = _bf16_bits(x[:16])
    hi = _bf16_bits(x[16:])
    w = lo | (hi << 16)
    o_ref[...] = jnp.concatenate(
        [w[:, q * GB:(q + 1) * GB].T for q in range(8)], axis=1)


def _linearize_half(et, h):
    return pl.pallas_call(
        _lin_half_body,
        grid=(NVB,),
        in_specs=[pl.BlockSpec((DH, VB), lambda i, h=h: (h, i))],
        out_specs=pl.BlockSpec((GB, 128), lambda i: (i, 0)),
        out_shape=jax.ShapeDtypeStruct((NVB * GB, 128), jnp.int32),
    )(et)


def _remap_ids(ids):
    t = ids.astype(jnp.int32)
    return (t & ~(VB - 1)) + 8 * (t & (GB - 1)) + ((t >> 10) & 7)


def _pool_body(art_hbm, sum_hbm, table_hbm, out_hbm,
               idx_a, idx_s, rows_v, pooled_v, sem):
    wid = lax.axis_index("s") * NC + lax.axis_index("c")
    base = wid * BPW
    pltpu.sync_copy(art_hbm.at[pl.ds(base, BPW)], idx_a)
    pltpu.sync_copy(sum_hbm.at[pl.ds(base, BPW)], idx_s)

    def gather(i, buf):
        return [
            pltpu.make_async_copy(
                table_hbm.at[idx_a.at[i, 0]],
                rows_v.at[buf, pl.ds(0, 100)], sem),
            pltpu.make_async_copy(
                table_hbm.at[idx_a.at[i, 1]],
                rows_v.at[buf, pl.ds(100, 100)], sem),
            pltpu.make_async_copy(
                table_hbm.at[idx_s.at[i]],
                rows_v.at[buf, pl.ds(200, LS)], sem),
        ]

    def gather_start(i, buf):
        for c in gather(i, buf):
            c.start()

    def gather_wait(buf):
        for c in gather(0, buf):
            c.wait()

    def reduce_store(i, buf):
        def red_body(r, accs):
            new = []
            for j in range(2):
                w = rows_v[buf, j * HALF + r, pl.ds(0, 16)]
                u0 = plsc.bitcast(w << 16, jnp.float32)
                u1 = plsc.bitcast(w & jnp.int32(-65536), jnp.float32)
                new.append(accs[j * 2] + u0)
                new.append(accs[j * 2 + 1] + u1)
            return tuple(new)

        accs = lax.fori_loop(
            0, HALF, red_body,
            tuple(jnp.zeros((16,), jnp.float32) for _ in range(4)))
        for db in range(2):
            pooled_v[i, pl.ds(db * 16, 16)] = (
                (accs[db] + accs[2 + db]) * (1.0 / L))

    gather_start(0, 0)

    def body(k, _):
        i0 = 2 * k
        gather_start(i0 + 1, 1)
        gather_wait(0)
        reduce_store(i0, 0)

        @pl.when(k < BPW // 2 - 1)
        def _():
            gather_start(i0 + 2, 0)

        gather_wait(1)
        reduce_store(i0 + 1, 1)
        return 0

    lax.fori_loop(0, BPW // 2, body, 0)
    pltpu.sync_copy(pooled_v, out_hbm.at[pl.ds(base, BPW)])


_pool = functools.partial(
    pl.kernel,
    mesh=plsc.VectorSubcoreMesh(core_axis_name="c", subcore_axis_name="s"),
    compiler_params=pltpu.CompilerParams(use_tc_tiling_on_sc=False,
                                         needs_layout_passes=False),
    out_type=jax.ShapeDtypeStruct((B, DH), jnp.float32),
    scratch_types=[
        pltpu.VMEM((BPW, 2, 100), jnp.int32),
        pltpu.VMEM((BPW, LS), jnp.int32),
        pltpu.VMEM((2, L, 16), jnp.int32),
        pltpu.VMEM((BPW, DH), jnp.float32),
        pltpu.SemaphoreType.DMA,
    ],
)(_pool_body)


def _mlp_body(x0_ref, x1_ref, w1a_ref, w1b_ref, b1_ref, w2_ref, b2_ref,
              o_ref):
    h = jnp.maximum(
        jnp.dot(x0_ref[...], w1a_ref[...],
                preferred_element_type=jnp.float32,
                precision=lax.Precision.HIGHEST)
        + jnp.dot(x1_ref[...], w1b_ref[...],
                  preferred_element_type=jnp.float32,
                  precision=lax.Precision.HIGHEST)
        + b1_ref[...], 0.0)
    z = jnp.dot(h, w2_ref[...], preferred_element_type=jnp.float32,
                precision=lax.Precision.HIGHEST) + b2_ref[...]
    o_ref[...] = jax.nn.sigmoid(z)


def kernel(article_ids, summary_ids, embedding, W1, b1, W2, b2):
    et = embedding.T
    art = _remap_ids(article_ids).reshape(B, 2, 100)
    summ = _remap_ids(summary_ids)
    tab0 = _linearize_half(et, 0).reshape(TAB_ROWS, 16)
    p0 = _pool(art, summ, tab0)
    tab1 = _linearize_half(et, 1).reshape(TAB_ROWS, 16)
    p1 = _pool(art, summ, tab1)
    out = pl.pallas_call(
        _mlp_body,
        out_shape=jax.ShapeDtypeStruct((B, 1), jnp.float32),
    )(p0, p1, W1[:DH], W1[DH:], b1.reshape(1, 128), W2, b2.reshape(1, 1))
    return out


# speedup vs baseline: 1.5838x; 1.5838x over previous
import functools

import jax
import jax.numpy as jnp
from jax import lax
from jax.experimental import pallas as pl
from jax.experimental.pallas import tpu as pltpu
from jax.experimental.pallas import tpu_sc as plsc

B = 4096
LA = 200
LS = 50
L = LA + LS
HALF = 125
D = 64
NC = 2
NS = 16
NW = NC * NS
BPW = B // NW

VB = 8192
NVB = 123
HB = 12
TAB_ROWS = NVB * VB


def _lin_body(et_ref, o_ref):
    x = et_ref[...]
    y0 = x[:, : VB // 2].T
    y1 = x[:, VB // 2:].T
    o_ref[...] = jnp.concatenate([y0, y1], axis=1)


def _linearize(et):
    return pl.pallas_call(
        _lin_body,
        grid=(NVB,),
        in_specs=[pl.BlockSpec((D, VB), lambda i: (0, i))],
        out_specs=pl.BlockSpec((VB // 2, 128), lambda i: (i, 0)),
        out_shape=jax.ShapeDtypeStruct((NVB * (VB // 2), 128), jnp.float32),
    )(et)


def _remap_ids(ids):
    t = ids.astype(jnp.int32)
    return (t & ~(VB - 1)) + 2 * (t & (VB // 2 - 1)) + ((t >> HB) & 1)


def _pool_body(art_hbm, sum_hbm, table_hbm, out_hbm,
               idx_a, idx_s, rows_v, pooled_v, sem):
    wid = lax.axis_index("s") * NC + lax.axis_index("c")
    base = wid * BPW
    pltpu.sync_copy(art_hbm.at[pl.ds(base, BPW)], idx_a)
    pltpu.sync_copy(sum_hbm.at[pl.ds(base, BPW)], idx_s)

    def gather(i, buf):
        return [
            pltpu.make_async_copy(
                table_hbm.at[idx_a.at[i, 0]],
                rows_v.at[buf, pl.ds(0, 100)], sem),
            pltpu.make_async_copy(
                table_hbm.at[idx_a.at[i, 1]],
                rows_v.at[buf, pl.ds(100, 100)], sem),
            pltpu.make_async_copy(
                table_hbm.at[idx_s.at[i]],
                rows_v.at[buf, pl.ds(200, LS)], sem),
        ]

    def gather_start(i, buf):
        for c in gather(i, buf):
            c.start()

    def gather_wait(buf):
        for c in gather(0, buf):
            c.wait()

    def reduce_store(i, buf):
        def red_body(r, accs):
            new = []
            for j in range(2):
                for db in range(4):
                    new.append(accs[j * 4 + db]
                               + rows_v[buf, j * HALF + r, pl.ds(db * 16, 16)])
            return tuple(new)

        accs = lax.fori_loop(
            0, HALF, red_body,
            tuple(jnp.zeros((16,), jnp.float32) for _ in range(8)),
            unroll=5)
        for db in range(4):
            pooled_v[i, pl.ds(db * 16, 16)] = (
                (accs[db] + accs[4 + db]) * (1.0 / L))

    gather_start(0, 0)

    def body(k, _):
        i0 = 2 * k
        gather_start(i0 + 1, 1)
        gather_wait(0)
        reduce_store(i0, 0)

        @pl.when(k < BPW // 2 - 1)
        def _():
            gather_start(i0 + 2, 0)

        gather_wait(1)
        reduce_store(i0 + 1, 1)
        return 0

    lax.fori_loop(0, BPW // 2, body, 0)
    pltpu.sync_copy(pooled_v, out_hbm.at[pl.ds(base, BPW)])


_pool = functools.partial(
    pl.kernel,
    mesh=plsc.VectorSubcoreMesh(core_axis_name="c", subcore_axis_name="s"),
    compiler_params=pltpu.CompilerParams(use_tc_tiling_on_sc=False),
    out_type=jax.ShapeDtypeStruct((B, D), jnp.float32),
    scratch_types=[
        pltpu.VMEM((BPW, 2, 100), jnp.int32),
        pltpu.VMEM((BPW, LS), jnp.int32),
        pltpu.VMEM((2, L, D), jnp.float32),
        pltpu.VMEM((BPW, D), jnp.float32),
        pltpu.SemaphoreType.DMA,
    ],
)(_pool_body)


def _mlp_body(x_ref, w1_ref, b1_ref, w2_ref, b2_ref, o_ref):
    x = x_ref[...]
    h = jnp.maximum(
        jnp.dot(x, w1_ref[...], preferred_element_type=jnp.float32,
                precision=lax.Precision.HIGHEST) + b1_ref[...], 0.0)
    z = jnp.dot(h, w2_ref[...], preferred_element_type=jnp.float32,
                precision=lax.Precision.HIGHEST) + b2_ref[...]
    o_ref[...] = jax.nn.sigmoid(z)


def kernel(article_ids, summary_ids, embedding, W1, b1, W2, b2):
    tab = _linearize(embedding.T).reshape(TAB_ROWS, D)
    pooled = _pool(_remap_ids(article_ids).reshape(B, 2, 100),
                   _remap_ids(summary_ids), tab)
    out = pl.pallas_call(
        _mlp_body,
        out_shape=jax.ShapeDtypeStruct((B, 1), jnp.float32),
    )(pooled, W1, b1.reshape(1, 128), W2, b2.reshape(1, 1))
    return out


# speedup vs baseline: 1.7089x; 1.0790x over previous
import functools

import jax
import jax.numpy as jnp
from jax import lax
from jax.experimental import pallas as pl
from jax.experimental.pallas import tpu as pltpu
from jax.experimental.pallas import tpu_sc as plsc

B = 4096
LA = 200
LS = 50
L = LA + LS
HALF = 125
D = 64
NC = 2
NS = 16
NW = NC * NS
BPW = B // NW

VB = 16384
NVB = 62
HB = 13
TAB_ROWS = NVB * VB


def _lin_body(et_ref, o_ref):
    x = et_ref[...]
    y0 = x[:, : VB // 2].T
    y1 = x[:, VB // 2:].T
    o_ref[...] = jnp.concatenate([y0, y1], axis=1)


def _linearize(et):
    return pl.pallas_call(
        _lin_body,
        grid=(NVB,),
        in_specs=[pl.BlockSpec((D, VB), lambda i: (0, i))],
        out_specs=pl.BlockSpec((VB // 2, 128), lambda i: (i, 0)),
        out_shape=jax.ShapeDtypeStruct((NVB * (VB // 2), 128), jnp.float32),
    )(et)


def _remap_ids(ids):
    t = ids.astype(jnp.int32)
    return (t & ~(VB - 1)) + 2 * (t & (VB // 2 - 1)) + ((t >> HB) & 1)


def _pool_body(art_hbm, sum_hbm, table_hbm, out_hbm,
               idx_a, idx_s, rows_v, pooled_v, sem):
    wid = lax.axis_index("s") * NC + lax.axis_index("c")
    base = wid * BPW
    pltpu.sync_copy(art_hbm.at[pl.ds(base, BPW)], idx_a)
    pltpu.sync_copy(sum_hbm.at[pl.ds(base, BPW)], idx_s)

    def gather(i, buf):
        return [
            pltpu.make_async_copy(
                table_hbm.at[idx_a.at[i, 0]],
                rows_v.at[buf, pl.ds(0, 100)], sem),
            pltpu.make_async_copy(
                table_hbm.at[idx_a.at[i, 1]],
                rows_v.at[buf, pl.ds(100, 100)], sem),
            pltpu.make_async_copy(
                table_hbm.at[idx_s.at[i]],
                rows_v.at[buf, pl.ds(200, LS)], sem),
        ]

    def gather_start(i, buf):
        for c in gather(i, buf):
            c.start()

    def gather_wait(buf):
        for c in gather(0, buf):
            c.wait()

    def reduce_store(i, buf):
        def red_body(r, accs):
            new = []
            for j in range(2):
                for db in range(4):
                    new.append(accs[j * 4 + db]
                               + rows_v[buf, j * HALF + r, pl.ds(db * 16, 16)])
            return tuple(new)

        accs = lax.fori_loop(
            0, HALF, red_body,
            tuple(jnp.zeros((16,), jnp.float32) for _ in range(8)),
            unroll=5)
        for db in range(4):
            pooled_v[i, pl.ds(db * 16, 16)] = (
                (accs[db] + accs[4 + db]) * (1.0 / L))

    gather_start(0, 0)

    def body(k, _):
        i0 = 2 * k
        gather_start(i0 + 1, 1)
        gather_wait(0)
        reduce_store(i0, 0)

        @pl.when(k < BPW // 2 - 1)
        def _():
            gather_start(i0 + 2, 0)

        gather_wait(1)
        reduce_store(i0 + 1, 1)
        return 0

    lax.fori_loop(0, BPW // 2, body, 0)
    pltpu.sync_copy(pooled_v, out_hbm.at[pl.ds(base, BPW)])


_pool = functools.partial(
    pl.kernel,
    mesh=plsc.VectorSubcoreMesh(core_axis_name="c", subcore_axis_name="s"),
    compiler_params=pltpu.CompilerParams(use_tc_tiling_on_sc=False),
    out_type=jax.ShapeDtypeStruct((B, D), jnp.float32),
    scratch_types=[
        pltpu.VMEM((BPW, 2, 100), jnp.int32),
        pltpu.VMEM((BPW, LS), jnp.int32),
        pltpu.VMEM((2, L, D), jnp.float32),
        pltpu.VMEM((BPW, D), jnp.float32),
        pltpu.SemaphoreType.DMA,
    ],
)(_pool_body)


def _mlp_body(x_ref, w1_ref, b1_ref, w2_ref, b2_ref, o_ref):
    x = x_ref[...]
    h = jnp.maximum(
        jnp.dot(x, w1_ref[...], preferred_element_type=jnp.float32,
                precision=lax.Precision.HIGHEST) + b1_ref[...], 0.0)
    z = jnp.dot(h, w2_ref[...], preferred_element_type=jnp.float32,
                precision=lax.Precision.HIGHEST) + b2_ref[...]
    o_ref[...] = jax.nn.sigmoid(z)


def kernel(article_ids, summary_ids, embedding, W1, b1, W2, b2):
    tab = _linearize(embedding.T).reshape(TAB_ROWS, D)
    pooled = _pool(_remap_ids(article_ids).reshape(B, 2, 100),
                   _remap_ids(summary_ids), tab)
    out = pl.pallas_call(
        _mlp_body,
        out_shape=jax.ShapeDtypeStruct((B, 1), jnp.float32),
    )(pooled, W1, b1.reshape(1, 128), W2, b2.reshape(1, 1))
    return out


# speedup vs baseline: 1.7718x; 1.0368x over previous
import functools

import jax
import jax.numpy as jnp
from jax import lax
from jax.experimental import pallas as pl
from jax.experimental.pallas import tpu as pltpu
from jax.experimental.pallas import tpu_sc as plsc

B = 4096
LA = 200
LS = 50
L = LA + LS
HALF = 125
D = 64
NC = 2
NS = 16
NW = NC * NS
BPW = B // NW

VB = 32768
NVB = 31
HB = 14
TAB_ROWS = NVB * VB


def _lin_body(et_ref, o_ref):
    x = et_ref[...]
    y0 = x[:, : VB // 2].T
    y1 = x[:, VB // 2:].T
    o_ref[...] = jnp.concatenate([y0, y1], axis=1)


def _linearize(et):
    return pl.pallas_call(
        _lin_body,
        grid=(NVB,),
        in_specs=[pl.BlockSpec((D, VB), lambda i: (0, i))],
        out_specs=pl.BlockSpec((VB // 2, 128), lambda i: (i, 0)),
        out_shape=jax.ShapeDtypeStruct((NVB * (VB // 2), 128), jnp.float32),
    )(et)


def _remap_ids(ids):
    t = ids.astype(jnp.int32)
    return (t & ~(VB - 1)) + 2 * (t & (VB // 2 - 1)) + ((t >> HB) & 1)


def _pool_body(art_hbm, sum_hbm, table_hbm, out_hbm,
               idx_a, idx_s, rows_v, pooled_v, sem):
    wid = lax.axis_index("s") * NC + lax.axis_index("c")
    base = wid * BPW
    pltpu.sync_copy(art_hbm.at[pl.ds(base, BPW)], idx_a)
    pltpu.sync_copy(sum_hbm.at[pl.ds(base, BPW)], idx_s)

    def gather(i, buf):
        return [
            pltpu.make_async_copy(
                table_hbm.at[idx_a.at[i, 0]],
                rows_v.at[buf, pl.ds(0, 100)], sem),
            pltpu.make_async_copy(
                table_hbm.at[idx_a.at[i, 1]],
                rows_v.at[buf, pl.ds(100, 100)], sem),
            pltpu.make_async_copy(
                table_hbm.at[idx_s.at[i]],
                rows_v.at[buf, pl.ds(200, LS)], sem),
        ]

    def gather_start(i, buf):
        for c in gather(i, buf):
            c.start()

    def gather_wait(buf):
        for c in gather(0, buf):
            c.wait()

    def reduce_store(i, buf):
        def red_body(r, accs):
            new = []
            for j in range(2):
                for db in range(4):
                    new.append(accs[j * 4 + db]
                               + rows_v[buf, j * HALF + r, pl.ds(db * 16, 16)])
            return tuple(new)

        accs = lax.fori_loop(
            0, HALF, red_body,
            tuple(jnp.zeros((16,), jnp.float32) for _ in range(8)),
            unroll=5)
        for db in range(4):
            pooled_v[i, pl.ds(db * 16, 16)] = (
                (accs[db] + accs[4 + db]) * (1.0 / L))

    gather_start(0, 0)

    def body(k, _):
        i0 = 2 * k
        gather_start(i0 + 1, 1)
        gather_wait(0)
        reduce_store(i0, 0)

        @pl.when(k < BPW // 2 - 1)
        def _():
            gather_start(i0 + 2, 0)

        gather_wait(1)
        reduce_store(i0 + 1, 1)
        return 0

    lax.fori_loop(0, BPW // 2, body, 0)
    pltpu.sync_copy(pooled_v, out_hbm.at[pl.ds(base, BPW)])


_pool = functools.partial(
    pl.kernel,
    mesh=plsc.VectorSubcoreMesh(core_axis_name="c", subcore_axis_name="s"),
    compiler_params=pltpu.CompilerParams(use_tc_tiling_on_sc=False),
    out_type=jax.ShapeDtypeStruct((B, D), jnp.float32),
    scratch_types=[
        pltpu.VMEM((BPW, 2, 100), jnp.int32),
        pltpu.VMEM((BPW, LS), jnp.int32),
        pltpu.VMEM((2, L, D), jnp.float32),
        pltpu.VMEM((BPW, D), jnp.float32),
        pltpu.SemaphoreType.DMA,
    ],
)(_pool_body)


def _mlp_body(x_ref, w1_ref, b1_ref, w2_ref, b2_ref, o_ref):
    x = x_ref[...]
    h = jnp.maximum(
        jnp.dot(x, w1_ref[...], preferred_element_type=jnp.float32,
                precision=lax.Precision.HIGHEST) + b1_ref[...], 0.0)
    z = jnp.dot(h, w2_ref[...], preferred_element_type=jnp.float32,
                precision=lax.Precision.HIGHEST) + b2_ref[...]
    o_ref[...] = jax.nn.sigmoid(z)


def kernel(article_ids, summary_ids, embedding, W1, b1, W2, b2):
    tab = _linearize(embedding.T).reshape(TAB_ROWS, D)
    pooled = _pool(_remap_ids(article_ids).reshape(B, 2, 100),
                   _remap_ids(summary_ids), tab)
    out = pl.pallas_call(
        _mlp_body,
        out_shape=jax.ShapeDtypeStruct((B, 1), jnp.float32),
    )(pooled, W1, b1.reshape(1, 128), W2, b2.reshape(1, 1))
    return out
